# baseline (speedup 1.0000x reference)
import functools

import jax
import jax.numpy as jnp
from jax import lax
from jax.experimental import pallas as pl
from jax.experimental.pallas import tpu as pltpu
from jax.experimental.pallas import tpu_sc as plsc

VOCAB = 1000000
EMBED_DIM = 64
SEQ_LEN = 200
BATCH = 4096

_NC = 2
_NS = 16
_NW = _NC * _NS

_CH = 256
_SLICE = 31264
_SLICE_LAST = VOCAB - 31 * _SLICE
_NPAIR = 61
_NPAIR_LAST = 60
_TAIL = _SLICE - 122 * _CH
_TAIL_LAST = _SLICE_LAST - 120 * _CH


def _rm_compute(buf, ob, ngroups):
    def grp(g, rows):
        acc = jnp.zeros((16,), jnp.float32)
        for d in range(EMBED_DIM):
            col = jnp.full((16,), d, jnp.int32)
            acc = acc + plsc.load_gather(buf, [rows, col])
        ob[pl.ds(g * 16, 16)] = acc * (1.0 / EMBED_DIM)
        return rows + 16

    lax.fori_loop(0, ngroups, grp, lax.iota(jnp.int32, 16))


def _rowmean_body(tab_hbm, rm_hbm, buf0, buf1, ob0, ob1, sem0, sem1):
    wid = lax.axis_index("s") * _NC + lax.axis_index("c")
    v0 = wid * _SLICE

    def cstart(c, buf, sem):
        pltpu.make_async_copy(
            tab_hbm.at[pl.ds(v0 + c * _CH, _CH), :], buf, sem).start()

    def cwait(buf, sem):
        pltpu.make_async_copy(
            tab_hbm.at[pl.ds(0, _CH), :], buf, sem).wait()

    is_last = wid == _NW - 1
    npair = jnp.where(is_last, _NPAIR_LAST, _NPAIR)
    nfull = 2 * npair

    cstart(0, buf0, sem0)
    cstart(1, buf1, sem1)

    def pair(j, carry):
        c0 = 2 * j
        for (buf, ob, sem, c) in ((buf0, ob0, sem0, c0),
                                  (buf1, ob1, sem1, c0 + 1)):
            cwait(buf, sem)
            _rm_compute(buf, ob, _CH // 16)
            pltpu.sync_copy(ob, rm_hbm.at[pl.ds(v0 + c * _CH, _CH)])

            @pl.when(c + 2 < nfull)
            def _():
                cstart(c + 2, buf, sem)
        return carry

    lax.fori_loop(0, npair, pair, 0)

    @pl.when(jnp.logical_not(is_last))
    def _():
        pltpu.make_async_copy(
            tab_hbm.at[pl.ds(v0 + 122 * _CH, _TAIL), :],
            buf0.at[pl.ds(0, _TAIL), :], sem0).start()
        pltpu.make_async_copy(
            tab_hbm.at[pl.ds(0, _TAIL), :],
            buf0.at[pl.ds(0, _TAIL), :], sem0).wait()
        _rm_compute(buf0, ob0, _TAIL // 16)
        pltpu.sync_copy(ob0.at[pl.ds(0, _TAIL)],
                        rm_hbm.at[pl.ds(v0 + 122 * _CH, _TAIL)])

    @pl.when(is_last)
    def _():
        pltpu.make_async_copy(
            tab_hbm.at[pl.ds(v0 + 120 * _CH, _TAIL_LAST), :],
            buf0.at[pl.ds(0, _TAIL_LAST), :], sem0).start()
        pltpu.make_async_copy(
            tab_hbm.at[pl.ds(0, _TAIL_LAST), :],
            buf0.at[pl.ds(0, _TAIL_LAST), :], sem0).wait()
        _rm_compute(buf0, ob0, _TAIL_LAST // 16)
        pltpu.sync_copy(ob0.at[pl.ds(0, _TAIL_LAST)],
                        rm_hbm.at[pl.ds(v0 + 120 * _CH, _TAIL_LAST)])


def _rowmean(table):
    mesh = plsc.VectorSubcoreMesh(core_axis_name="c", subcore_axis_name="s")
    f = functools.partial(
        pl.kernel,
        mesh=mesh,
        compiler_params=pltpu.CompilerParams(needs_layout_passes=False),
        out_type=jax.ShapeDtypeStruct((VOCAB,), jnp.float32),
        scratch_types=[
            pltpu.VMEM((_CH, EMBED_DIM), jnp.float32),
            pltpu.VMEM((_CH, EMBED_DIM), jnp.float32),
            pltpu.VMEM((_CH,), jnp.float32),
            pltpu.VMEM((_CH,), jnp.float32),
            pltpu.SemaphoreType.DMA,
            pltpu.SemaphoreType.DMA,
        ],
    )(_rowmean_body)
    return f(table)


_N_IDX = BATCH * SEQ_LEN
_CHUNK = _N_IDX // _NW


def _gather_body(idx_hbm, rm_hbm, out_hbm, idx_v, val_v, sem):
    wid = lax.axis_index("s") * _NC + lax.axis_index("c")
    base = wid * _CHUNK
    pltpu.sync_copy(idx_hbm.at[pl.ds(base, _CHUNK)], idx_v)
    pltpu.async_copy(rm_hbm.at[idx_v], val_v, sem).wait()
    pltpu.sync_copy(val_v, out_hbm.at[pl.ds(base, _CHUNK)])


def _gather(idx_flat, rowmean):
    mesh = plsc.VectorSubcoreMesh(core_axis_name="c", subcore_axis_name="s")
    f = functools.partial(
        pl.kernel,
        mesh=mesh,
        out_type=jax.ShapeDtypeStruct((_N_IDX,), jnp.float32),
        scratch_types=[
            pltpu.VMEM((_CHUNK,), jnp.int32),
            pltpu.VMEM((_CHUNK,), jnp.float32),
            pltpu.SemaphoreType.DMA,
        ],
    )(_gather_body)
    return f(idx_flat, rowmean)


def _head_body(m_ref, w_ref, b_ref, out_ref):
    z = jnp.sum(m_ref[...] * w_ref[...], axis=1) + b_ref[0]
    out_ref[...] = 1.0 / (1.0 + jnp.exp(-z))


def _head(m, W, b):
    return pl.pallas_call(
        _head_body,
        in_specs=[
            pl.BlockSpec((BATCH, SEQ_LEN), lambda: (0, 0)),
            pl.BlockSpec((1, SEQ_LEN), lambda: (0, 0)),
            pl.BlockSpec(memory_space=pltpu.SMEM),
        ],
        out_specs=pl.BlockSpec((BATCH,), lambda: (0,)),
        out_shape=jax.ShapeDtypeStruct((BATCH,), jnp.float32),
    )(m, W, b)


def kernel(x, table, W, b):
    rowmean = _rowmean(table)
    m = _gather(x.reshape(-1), rowmean)
    return _head(m.reshape(BATCH, SEQ_LEN), W, b)


# speedup vs baseline: 1.3881x; 1.3881x over previous
import functools

import jax
import jax.numpy as jnp
from jax import lax
from jax.experimental import pallas as pl
from jax.experimental.pallas import tpu as pltpu
from jax.experimental.pallas import tpu_sc as plsc

VOCAB = 1000000
EMBED_DIM = 64
SEQ_LEN = 200
BATCH = 4096

_NC = 2
_NS = 16
_NW = _NC * _NS

_XTC = 614400
_V2TC = _XTC // 2
_RBT = 4096
_GRIDT = _V2TC // _RBT
_TPBT = _RBT // 128


def _remap_body(x_ref, o_ref):
    v = x_ref[...]
    shuf = (v & jnp.int32(-256)) | ((v & 1) << 7) | ((v >> 1) & 127)
    o_ref[...] = jnp.where(v < _XTC, shuf, v)


def _remap(x):
    return pl.pallas_call(
        _remap_body,
        in_specs=[pl.BlockSpec((BATCH, SEQ_LEN), lambda: (0, 0))],
        out_specs=pl.BlockSpec((BATCH, SEQ_LEN), lambda: (0, 0)),
        out_shape=jax.ShapeDtypeStruct((BATCH, SEQ_LEN), jnp.int32),
    )(x)


def _rowmean_tc_body(tab_ref, out_ref):
    a = tab_ref[...]
    lane = lax.broadcasted_iota(jnp.int32, (128, 128), 0)
    col = lax.broadcasted_iota(jnp.int32, (128, 128), 1)
    par = jnp.where((lane // 64) == (col % 2), 1.0 / EMBED_DIM, 0.0)
    z = lax.dot_general(a, par.astype(jnp.float32), (((1,), (0,)), ((), ())),
                        precision=lax.Precision.HIGHEST,
                        preferred_element_type=jnp.float32)
    z3 = z.reshape(_TPBT, 128, 128)
    t = jnp.swapaxes(z3, 1, 2)
    out_ref[...] = t[:, 0:2, :]


def _rowmean_tc(table2):
    return pl.pallas_call(
        _rowmean_tc_body,
        grid=(_GRIDT,),
        in_specs=[pl.BlockSpec((_RBT, 128), lambda i: (i, 0))],
        out_specs=pl.BlockSpec((_TPBT, 2, 128), lambda i: (i, 0, 0)),
        out_shape=jax.ShapeDtypeStruct((_GRIDT * _TPBT, 2, 128), jnp.float32),
    )(table2)


_NSC = VOCAB - _XTC
_CH = 256
_SLC = 12288
_SLC_LAST = _NSC - 31 * _SLC
_NPAIR = 24
_NPAIR_LAST = 9
_TAIL_LAST = _SLC_LAST - 18 * _CH


def _rm_compute(buf, ob, ngroups):
    def grp(g, rows):
        acc = jnp.zeros((16,), jnp.float32)
        for d in range(EMBED_DIM):
            col = jnp.full((16,), d, jnp.int32)
            acc = acc + plsc.load_gather(buf, [rows, col])
        ob[pl.ds(g * 16, 16)] = acc * (1.0 / EMBED_DIM)
        return rows + 16

    lax.fori_loop(0, ngroups, grp, lax.iota(jnp.int32, 16))


def _rowmean_sc_body(tab_hbm, rm_hbm, buf0, buf1, ob0, ob1, sem0, sem1):
    wid = lax.axis_index("s") * _NC + lax.axis_index("c")
    v0 = wid * _SLC

    def cstart(c, buf, sem):
        pltpu.make_async_copy(
            tab_hbm.at[pl.ds(v0 + c * _CH, _CH), :], buf, sem).start()

    def cwait(buf, sem):
        pltpu.make_async_copy(
            tab_hbm.at[pl.ds(0, _CH), :], buf, sem).wait()

    is_last = wid == _NW - 1
    npair = jnp.where(is_last, _NPAIR_LAST, _NPAIR)
    nfull = 2 * npair

    cstart(0, buf0, sem0)
    cstart(1, buf1, sem1)

    def pair(j, carry):
        c0 = 2 * j
        for (buf, ob, sem, c) in ((buf0, ob0, sem0, c0),
                                  (buf1, ob1, sem1, c0 + 1)):
            cwait(buf, sem)
            _rm_compute(buf, ob, _CH // 16)
            pltpu.sync_copy(ob, rm_hbm.at[pl.ds(v0 + c * _CH, _CH)])

            @pl.when(c + 2 < nfull)
            def _():
                cstart(c + 2, buf, sem)
        return carry

    lax.fori_loop(0, npair, pair, 0)

    @pl.when(is_last)
    def _():
        pltpu.make_async_copy(
            tab_hbm.at[pl.ds(v0 + 18 * _CH, _TAIL_LAST), :],
            buf0.at[pl.ds(0, _TAIL_LAST), :], sem0).start()
        pltpu.make_async_copy(
            tab_hbm.at[pl.ds(0, _TAIL_LAST), :],
            buf0.at[pl.ds(0, _TAIL_LAST), :], sem0).wait()
        _rm_compute(buf0, ob0, _TAIL_LAST // 16)
        pltpu.sync_copy(ob0.at[pl.ds(0, _TAIL_LAST)],
                        rm_hbm.at[pl.ds(v0 + 18 * _CH, _TAIL_LAST)])


def _rowmean_sc(table_sc):
    mesh = plsc.VectorSubcoreMesh(core_axis_name="c", subcore_axis_name="s")
    f = functools.partial(
        pl.kernel,
        mesh=mesh,
        compiler_params=pltpu.CompilerParams(needs_layout_passes=False),
        out_type=jax.ShapeDtypeStruct((_NSC,), jnp.float32),
        scratch_types=[
            pltpu.VMEM((_CH, EMBED_DIM), jnp.float32),
            pltpu.VMEM((_CH, EMBED_DIM), jnp.float32),
            pltpu.VMEM((_CH,), jnp.float32),
            pltpu.VMEM((_CH,), jnp.float32),
            pltpu.SemaphoreType.DMA,
            pltpu.SemaphoreType.DMA,
        ],
    )(_rowmean_sc_body)
    return f(table_sc)


_N_IDX = BATCH * SEQ_LEN
_CHUNK = _N_IDX // _NW


def _gather_body(idx_hbm, rm_hbm, out_hbm, idx_v, val_v, sem):
    wid = lax.axis_index("s") * _NC + lax.axis_index("c")
    base = wid * _CHUNK
    pltpu.sync_copy(idx_hbm.at[pl.ds(base, _CHUNK)], idx_v)
    pltpu.async_copy(rm_hbm.at[idx_v], val_v, sem).wait()
    pltpu.sync_copy(val_v, out_hbm.at[pl.ds(base, _CHUNK)])


def _gather(idx_flat, rowmean):
    mesh = plsc.VectorSubcoreMesh(core_axis_name="c", subcore_axis_name="s")
    f = functools.partial(
        pl.kernel,
        mesh=mesh,
        out_type=jax.ShapeDtypeStruct((_N_IDX,), jnp.float32),
        scratch_types=[
            pltpu.VMEM((_CHUNK,), jnp.int32),
            pltpu.VMEM((_CHUNK,), jnp.float32),
            pltpu.SemaphoreType.DMA,
        ],
    )(_gather_body)
    return f(idx_flat, rowmean)


def _head_body(m_ref, w_ref, b_ref, out_ref):
    z = jnp.sum(m_ref[...] * w_ref[...], axis=1) + b_ref[0]
    out_ref[...] = 1.0 / (1.0 + jnp.exp(-z))


def _head(m, W, b):
    return pl.pallas_call(
        _head_body,
        in_specs=[
            pl.BlockSpec((BATCH, SEQ_LEN), lambda: (0, 0)),
            pl.BlockSpec((1, SEQ_LEN), lambda: (0, 0)),
            pl.BlockSpec(memory_space=pltpu.SMEM),
        ],
        out_specs=pl.BlockSpec((BATCH,), lambda: (0,)),
        out_shape=jax.ShapeDtypeStruct((BATCH,), jnp.float32),
    )(m, W, b)


def kernel(x, table, W, b):
    xp = _remap(x)
    rm_tc = _rowmean_tc(table[:_XTC].reshape(_V2TC, 128)).reshape(-1)
    rm_sc = _rowmean_sc(table[_XTC:])
    rowmean = jnp.concatenate([rm_tc, rm_sc])
    m = _gather(xp.reshape(-1), rowmean)
    return _head(m.reshape(BATCH, SEQ_LEN), W, b)


# speedup vs baseline: 1.4187x; 1.0221x over previous
import functools

import jax
import jax.numpy as jnp
from jax import lax
from jax.experimental import pallas as pl
from jax.experimental.pallas import tpu as pltpu
from jax.experimental.pallas import tpu_sc as plsc

VOCAB = 1000000
EMBED_DIM = 64
SEQ_LEN = 200
BATCH = 4096

_NC = 2
_NS = 16
_NW = _NC * _NS

_XTC = 548864
_V2TC = _XTC // 2
_RBT = 4096
_GRIDT = _V2TC // _RBT
_TPBT = _RBT // 128


def _remap_body(x_ref, o_ref):
    v = x_ref[...]
    shuf = (v & jnp.int32(-256)) | ((v & 1) << 7) | ((v >> 1) & 127)
    o_ref[...] = jnp.where(v < _XTC, shuf, v)


def _remap(x):
    return pl.pallas_call(
        _remap_body,
        in_specs=[pl.BlockSpec((BATCH, SEQ_LEN), lambda: (0, 0))],
        out_specs=pl.BlockSpec((BATCH, SEQ_LEN), lambda: (0, 0)),
        out_shape=jax.ShapeDtypeStruct((BATCH, SEQ_LEN), jnp.int32),
    )(x)


def _rowmean_tc_body(tab_ref, out_ref):
    a = tab_ref[...]
    lane = lax.broadcasted_iota(jnp.int32, (128, 128), 0)
    col = lax.broadcasted_iota(jnp.int32, (128, 128), 1)
    par = jnp.where((lane // 64) == (col % 2), 1.0 / EMBED_DIM, 0.0)
    z = lax.dot_general(a, par.astype(jnp.float32), (((1,), (0,)), ((), ())),
                        precision=lax.Precision.HIGHEST,
                        preferred_element_type=jnp.float32)
    z3 = z.reshape(_TPBT, 128, 128)
    t = jnp.swapaxes(z3, 1, 2)
    out_ref[...] = t[:, 0:2, :]


def _rowmean_tc(table2):
    return pl.pallas_call(
        _rowmean_tc_body,
        grid=(_GRIDT,),
        in_specs=[pl.BlockSpec((_RBT, 128), lambda i: (i, 0))],
        out_specs=pl.BlockSpec((_TPBT, 2, 128), lambda i: (i, 0, 0)),
        out_shape=jax.ShapeDtypeStruct((_GRIDT * _TPBT, 2, 128), jnp.float32),
    )(table2)


_NSC = VOCAB - _XTC
_CH = 256
_SLC = 14336
_SLC_LAST = _NSC - 31 * _SLC
_NPAIR = 28
_NPAIR_LAST = 13
_TAIL_LAST = _SLC_LAST - 26 * _CH


def _rm_compute(buf, ob, ngroups):
    def grp(g, rows):
        acc = jnp.zeros((16,), jnp.float32)
        for d in range(EMBED_DIM):
            col = jnp.full((16,), d, jnp.int32)
            acc = acc + plsc.load_gather(buf, [rows, col])
        ob[pl.ds(g * 16, 16)] = acc * (1.0 / EMBED_DIM)
        return rows + 16

    lax.fori_loop(0, ngroups, grp, lax.iota(jnp.int32, 16))


def _rowmean_sc_body(tab_hbm, rm_hbm, buf0, buf1, ob0, ob1, sem0, sem1):
    wid = lax.axis_index("s") * _NC + lax.axis_index("c")
    v0 = wid * _SLC

    def cstart(c, buf, sem):
        pltpu.make_async_copy(
            tab_hbm.at[pl.ds(v0 + c * _CH, _CH), :], buf, sem).start()

    def cwait(buf, sem):
        pltpu.make_async_copy(
            tab_hbm.at[pl.ds(0, _CH), :], buf, sem).wait()

    is_last = wid == _NW - 1
    npair = jnp.where(is_last, _NPAIR_LAST, _NPAIR)
    nfull = 2 * npair

    cstart(0, buf0, sem0)
    cstart(1, buf1, sem1)

    def pair(j, carry):
        c0 = 2 * j
        for (buf, ob, sem, c) in ((buf0, ob0, sem0, c0),
                                  (buf1, ob1, sem1, c0 + 1)):
            cwait(buf, sem)
            _rm_compute(buf, ob, _CH // 16)
            pltpu.sync_copy(ob, rm_hbm.at[pl.ds(v0 + c * _CH, _CH)])

            @pl.when(c + 2 < nfull)
            def _():
                cstart(c + 2, buf, sem)
        return carry

    lax.fori_loop(0, npair, pair, 0)

    @pl.when(is_last)
    def _():
        pltpu.make_async_copy(
            tab_hbm.at[pl.ds(v0 + 26 * _CH, _TAIL_LAST), :],
            buf0.at[pl.ds(0, _TAIL_LAST), :], sem0).start()
        pltpu.make_async_copy(
            tab_hbm.at[pl.ds(0, _TAIL_LAST), :],
            buf0.at[pl.ds(0, _TAIL_LAST), :], sem0).wait()
        _rm_compute(buf0, ob0, _TAIL_LAST // 16)
        pltpu.sync_copy(ob0.at[pl.ds(0, _TAIL_LAST)],
                        rm_hbm.at[pl.ds(v0 + 26 * _CH, _TAIL_LAST)])


def _rowmean_sc(table_sc):
    mesh = plsc.VectorSubcoreMesh(core_axis_name="c", subcore_axis_name="s")
    f = functools.partial(
        pl.kernel,
        mesh=mesh,
        compiler_params=pltpu.CompilerParams(needs_layout_passes=False),
        out_type=jax.ShapeDtypeStruct((_NSC,), jnp.float32),
        scratch_types=[
            pltpu.VMEM((_CH, EMBED_DIM), jnp.float32),
            pltpu.VMEM((_CH, EMBED_DIM), jnp.float32),
            pltpu.VMEM((_CH,), jnp.float32),
            pltpu.VMEM((_CH,), jnp.float32),
            pltpu.SemaphoreType.DMA,
            pltpu.SemaphoreType.DMA,
        ],
    )(_rowmean_sc_body)
    return f(table_sc)


def _concat_body(a_ref, b_ref, out_ref):
    out_ref[pl.ds(0, _XTC)] = a_ref[...]
    out_ref[pl.ds(_XTC, _NSC)] = b_ref[...]


def _concat(rm_tc, rm_sc):
    return pl.pallas_call(
        _concat_body,
        in_specs=[pl.BlockSpec(memory_space=pltpu.VMEM),
                  pl.BlockSpec(memory_space=pltpu.VMEM)],
        out_specs=pl.BlockSpec(memory_space=pltpu.VMEM),
        out_shape=jax.ShapeDtypeStruct((VOCAB,), jnp.float32),
    )(rm_tc, rm_sc)


_N_IDX = BATCH * SEQ_LEN
_CHUNK = _N_IDX // _NW


def _gather_body(idx_hbm, rm_hbm, out_hbm, idx_v, val_v, sem):
    wid = lax.axis_index("s") * _NC + lax.axis_index("c")
    base = wid * _CHUNK
    pltpu.sync_copy(idx_hbm.at[pl.ds(base, _CHUNK)], idx_v)
    pltpu.async_copy(rm_hbm.at[idx_v], val_v, sem).wait()
    pltpu.sync_copy(val_v, out_hbm.at[pl.ds(base, _CHUNK)])


def _gather(idx_flat, rowmean):
    mesh = plsc.VectorSubcoreMesh(core_axis_name="c", subcore_axis_name="s")
    f = functools.partial(
        pl.kernel,
        mesh=mesh,
        out_type=jax.ShapeDtypeStruct((_N_IDX,), jnp.float32),
        scratch_types=[
            pltpu.VMEM((_CHUNK,), jnp.int32),
            pltpu.VMEM((_CHUNK,), jnp.float32),
            pltpu.SemaphoreType.DMA,
        ],
    )(_gather_body)
    return f(idx_flat, rowmean)


def _head_body(m_ref, w_ref, b_ref, out_ref):
    z = jnp.sum(m_ref[...] * w_ref[...], axis=1) + b_ref[0]
    out_ref[...] = 1.0 / (1.0 + jnp.exp(-z))


def _head(m, W, b):
    return pl.pallas_call(
        _head_body,
        in_specs=[
            pl.BlockSpec((BATCH, SEQ_LEN), lambda: (0, 0)),
            pl.BlockSpec((1, SEQ_LEN), lambda: (0, 0)),
            pl.BlockSpec(memory_space=pltpu.SMEM),
        ],
        out_specs=pl.BlockSpec((BATCH,), lambda: (0,)),
        out_shape=jax.ShapeDtypeStruct((BATCH,), jnp.float32),
    )(m, W, b)


def kernel(x, table, W, b):
    xp = _remap(x)
    rm_tc = _rowmean_tc(table[:_XTC].reshape(_V2TC, 128)).reshape(-1)
    rm_sc = _rowmean_sc(table[_XTC:])
    rowmean = _concat(rm_tc, rm_sc)
    m = _gather(xp.reshape(-1), rowmean)
    return _head(m.reshape(BATCH, SEQ_LEN), W, b)
